# Initial kernel scaffold; baseline (speedup 1.0000x reference)
import jax
import jax.numpy as jnp
from jax.experimental import pallas as pl


def kernel(dense_x, lS_i, emb_tables, bot_w0, bot_b0, bot_w1, bot_b1, bot_w2, bot_b2, top_w0, top_b0, top_w1, top_b1, top_w2, top_b2):
    raise NotImplementedError("write your pallas kernel here")


# speedup vs baseline: 2.1802x; 2.1802x over previous
import functools

import numpy as np
import jax
import jax.numpy as jnp
from jax import lax
from jax.experimental import pallas as pl
from jax.experimental.pallas import tpu as pltpu
from jax.experimental.pallas import tpu_sc as plsc

B = 4096
NFIELDS = 26
VOCAB = 100000
D = 32
NF1 = NFIELDS + 1
ZDIM = NF1 * NF1


def _sc_gather(table_flat, flat_idx):
    info = plsc.get_sparse_core_info()
    nc, ns = info.num_cores, info.num_subcores
    nw = nc * ns
    tot = flat_idx.shape[0]
    per_w = tot // nw
    ch = 128
    n_ch = per_w // ch
    half = n_ch // 2
    mesh = plsc.VectorSubcoreMesh(core_axis_name="c", subcore_axis_name="s")

    @functools.partial(
        pl.kernel,
        mesh=mesh,
        out_type=jax.ShapeDtypeStruct((tot, D), jnp.float32),
        scratch_types=[
            pltpu.VMEM((per_w,), jnp.int32),
            pltpu.VMEM((per_w, D), jnp.float32),
            pltpu.SemaphoreType.DMA,
        ],
        compiler_params=pltpu.CompilerParams(use_tc_tiling_on_sc=False),
    )
    def k(table_hbm, idx_hbm, out_hbm, idx_v, rows_v, sem):
        wid = lax.axis_index("s") * nc + lax.axis_index("c")
        base = wid * per_w
        pltpu.sync_copy(idx_hbm.at[pl.ds(base, per_w)], idx_v)
        for r in range(2):
            copies = [
                pltpu.async_copy(
                    table_hbm.at[idx_v.at[pl.ds(c * ch, ch)]],
                    rows_v.at[pl.ds(c * ch, ch)],
                    sem,
                )
                for c in range(r * half, (r + 1) * half)
            ]
            for cp in copies:
                cp.wait()
        pltpu.sync_copy(rows_v, out_hbm.at[pl.ds(base, per_w)])

    return k(table_flat, flat_idx)


def _tc_body(xt_ref, lyt_ref, bw0, bb0, bw1, bb1, bw2, bb2,
             w0x, w0z, tb0, tw1, tb1, tw2, tb2, out_ref):
    f32 = jnp.float32
    x = xt_ref[...]
    h = jnp.maximum(jnp.dot(bw0[...], x, preferred_element_type=f32) + bb0[...], 0.0)
    h = jnp.maximum(jnp.dot(bw1[...], h, preferred_element_type=f32) + bb1[...], 0.0)
    x3 = jnp.maximum(jnp.dot(bw2[...], h, preferred_element_type=f32) + bb2[...], 0.0)

    t2 = jnp.concatenate([x3, lyt_ref[...]], axis=0)
    bb = t2.shape[1]
    t3 = t2.reshape(NF1, D, bb)
    zrows = []
    for i in range(NF1):
        zrows.append(jnp.sum(t3 * t3[i][None], axis=1))
    zt = jnp.concatenate(zrows, axis=0)

    a = jnp.dot(w0x[...], x3, preferred_element_type=f32)
    a = a + jnp.dot(w0z[...], zt, preferred_element_type=f32) + tb0[...]
    a = jnp.maximum(a, 0.0)
    a = jnp.maximum(jnp.dot(tw1[...], a, preferred_element_type=f32) + tb1[...], 0.0)
    o = jnp.dot(tw2[...], a, preferred_element_type=f32) + tb2[...]
    out_ref[...] = 1.0 / (1.0 + jnp.exp(-o))


def _tc_fused(xt, lyt, bw0, bb0, bw1, bb1, bw2, bb2,
              w0x, w0z, tb0, tw1, tb1, tw2, tb2, block_b=512):
    nb = B // block_b

    def full(a):
        return pl.BlockSpec(a.shape, lambda b: (0,) * a.ndim)

    weights = (bw0, bb0, bw1, bb1, bw2, bb2, w0x, w0z, tb0, tw1, tb1, tw2, tb2)
    return pl.pallas_call(
        _tc_body,
        grid=(nb,),
        in_specs=[
            pl.BlockSpec((13, block_b), lambda b: (0, b)),
            pl.BlockSpec((NFIELDS * D, block_b), lambda b: (0, b)),
        ] + [full(w) for w in weights],
        out_specs=pl.BlockSpec((1, block_b), lambda b: (0, b)),
        out_shape=jax.ShapeDtypeStruct((1, B), jnp.float32),
    )(xt, lyt, *weights)


def kernel(dense_x, lS_i, emb_tables, bot_w0, bot_b0, bot_w1, bot_b1,
           bot_w2, bot_b2, top_w0, top_b0, top_w1, top_b1, top_w2, top_b2):
    offsets = (jnp.arange(NFIELDS, dtype=jnp.int32) * VOCAB)[:, None]
    flat_idx = (lS_i.astype(jnp.int32) + offsets).reshape(-1)
    table_flat = emb_tables.reshape(NFIELDS * VOCAB, D)

    ly_flat = _sc_gather(table_flat, flat_idx)

    lyt = ly_flat.reshape(NFIELDS, B, D).transpose(0, 2, 1).reshape(NFIELDS * D, B)
    xt = dense_x.T

    li, lj = np.tril_indices(NF1, -1)
    sel = np.asarray(li * NF1 + lj)
    w0x = top_w0[:, :D]
    w0z = jnp.zeros((top_w0.shape[0], ZDIM), jnp.float32).at[:, sel].set(top_w0[:, D:])

    def col(b):
        return b.reshape(-1, 1)

    out = _tc_fused(xt, lyt, bot_w0, col(bot_b0), bot_w1, col(bot_b1),
                    bot_w2, col(bot_b2), w0x, w0z, col(top_b0),
                    top_w1, col(top_b1), top_w2, col(top_b2))
    return out.reshape(B, 1)


# speedup vs baseline: 2.2162x; 1.0165x over previous
import functools

import numpy as np
import jax
import jax.numpy as jnp
from jax import lax
from jax.experimental import pallas as pl
from jax.experimental.pallas import tpu as pltpu
from jax.experimental.pallas import tpu_sc as plsc

B = 4096
NFIELDS = 26
VOCAB = 100000
D = 32
NF1 = NFIELDS + 1
ZDIM = NF1 * NF1


def _sc_gather(table_flat, flat_idx):
    info = plsc.get_sparse_core_info()
    nc, ns = info.num_cores, info.num_subcores
    nw = nc * ns
    tot = flat_idx.shape[0]
    per_w = tot // nw
    ch = 128
    n_ch = per_w // ch
    half = n_ch // 2
    mesh = plsc.VectorSubcoreMesh(core_axis_name="c", subcore_axis_name="s")

    @functools.partial(
        pl.kernel,
        mesh=mesh,
        out_type=jax.ShapeDtypeStruct((tot, D), jnp.float32),
        scratch_types=[
            pltpu.VMEM((per_w,), jnp.int32),
            pltpu.VMEM((per_w, D), jnp.float32),
            pltpu.SemaphoreType.DMA,
        ],
        compiler_params=pltpu.CompilerParams(use_tc_tiling_on_sc=False),
    )
    def k(table_hbm, idx_hbm, out_hbm, idx_v, rows_v, sem):
        wid = lax.axis_index("s") * nc + lax.axis_index("c")
        base = wid * per_w
        pltpu.sync_copy(idx_hbm.at[pl.ds(base, per_w)], idx_v)
        for r in range(2):
            copies = [
                pltpu.async_copy(
                    table_hbm.at[idx_v.at[pl.ds(c * ch, ch)]],
                    rows_v.at[pl.ds(c * ch, ch)],
                    sem,
                )
                for c in range(r * half, (r + 1) * half)
            ]
            for cp in copies:
                cp.wait()
        pltpu.sync_copy(rows_v, out_hbm.at[pl.ds(base, per_w)])

    return k(table_flat, flat_idx)


def _tc_body(xt_ref, lyt_ref, bw0, bb0, bw1, bb1, bw2, bb2,
             w0x, w0z, tb0, tw1, tb1, tw2, tb2, out_ref):
    f32 = jnp.float32
    x = xt_ref[...]
    h = jnp.maximum(jnp.dot(bw0[...], x, preferred_element_type=f32) + bb0[...], 0.0)
    h = jnp.maximum(jnp.dot(bw1[...], h, preferred_element_type=f32) + bb1[...], 0.0)
    x3 = jnp.maximum(jnp.dot(bw2[...], h, preferred_element_type=f32) + bb2[...], 0.0)

    lyb = lyt_ref[...]
    lyt = jnp.transpose(lyb, (0, 2, 1)).reshape(NFIELDS * D, lyb.shape[1])
    t2 = jnp.concatenate([x3, lyt], axis=0)
    bb = t2.shape[1]
    t3 = t2.reshape(NF1, D, bb)
    zrows = []
    for i in range(NF1):
        zrows.append(jnp.sum(t3 * t3[i][None], axis=1))
    zt = jnp.concatenate(zrows, axis=0)

    a = jnp.dot(w0x[...], x3, preferred_element_type=f32)
    a = a + jnp.dot(w0z[...], zt, preferred_element_type=f32) + tb0[...]
    a = jnp.maximum(a, 0.0)
    a = jnp.maximum(jnp.dot(tw1[...], a, preferred_element_type=f32) + tb1[...], 0.0)
    o = jnp.dot(tw2[...], a, preferred_element_type=f32) + tb2[...]
    out_ref[...] = 1.0 / (1.0 + jnp.exp(-o))


def _tc_fused(xt, lyt, bw0, bb0, bw1, bb1, bw2, bb2,
              w0x, w0z, tb0, tw1, tb1, tw2, tb2, block_b=512):
    nb = B // block_b

    def full(a):
        return pl.BlockSpec(a.shape, lambda b: (0,) * a.ndim)

    weights = (bw0, bb0, bw1, bb1, bw2, bb2, w0x, w0z, tb0, tw1, tb1, tw2, tb2)
    return pl.pallas_call(
        _tc_body,
        grid=(nb,),
        in_specs=[
            pl.BlockSpec((13, block_b), lambda b: (0, b)),
            pl.BlockSpec((NFIELDS, block_b, D), lambda b: (0, b, 0)),
        ] + [full(w) for w in weights],
        out_specs=pl.BlockSpec((1, block_b), lambda b: (0, b)),
        out_shape=jax.ShapeDtypeStruct((1, B), jnp.float32),
    )(xt, lyt, *weights)


def kernel(dense_x, lS_i, emb_tables, bot_w0, bot_b0, bot_w1, bot_b1,
           bot_w2, bot_b2, top_w0, top_b0, top_w1, top_b1, top_w2, top_b2):
    offsets = (jnp.arange(NFIELDS, dtype=jnp.int32) * VOCAB)[:, None]
    flat_idx = (lS_i.astype(jnp.int32) + offsets).reshape(-1)
    table_flat = emb_tables.reshape(NFIELDS * VOCAB, D)

    ly_flat = _sc_gather(table_flat, flat_idx)

    ly3 = ly_flat.reshape(NFIELDS, B, D)
    xt = dense_x.T

    li, lj = np.tril_indices(NF1, -1)
    sel = np.asarray(li * NF1 + lj)
    w0x = top_w0[:, :D]
    w0z = jnp.zeros((top_w0.shape[0], ZDIM), jnp.float32).at[:, sel].set(top_w0[:, D:])

    def col(b):
        return b.reshape(-1, 1)

    out = _tc_fused(xt, ly3, bot_w0, col(bot_b0), bot_w1, col(bot_b1),
                    bot_w2, col(bot_b2), w0x, w0z, col(top_b0),
                    top_w1, col(top_b1), top_w2, col(top_b2))
    return out.reshape(B, 1)


# speedup vs baseline: 10.8180x; 4.8813x over previous
import functools

import numpy as np
import jax
import jax.numpy as jnp
from jax import lax
from jax.experimental import pallas as pl
from jax.experimental.pallas import tpu as pltpu
from jax.experimental.pallas import tpu_sc as plsc

B = 4096
NFIELDS = 26
VOCAB = 100000
D = 32
NF1 = NFIELDS + 1
ZDIM = NF1 * NF1


def _sc_gather(table_t, idx2):
    info = plsc.get_sparse_core_info()
    nc, ns = info.num_cores, info.num_subcores
    nw = nc * ns
    rows = table_t.shape[0]
    per_w = rows // nw
    ngrp = B // 16
    mesh = plsc.VectorSubcoreMesh(core_axis_name="c", subcore_axis_name="s")

    @functools.partial(
        pl.kernel,
        mesh=mesh,
        out_type=jax.ShapeDtypeStruct((rows, B), jnp.float32),
        scratch_types=[
            pltpu.VMEM((VOCAB,), jnp.float32),
            pltpu.VMEM((B,), jnp.int32),
            pltpu.VMEM((B,), jnp.float32),
        ],
        compiler_params=pltpu.CompilerParams(needs_layout_passes=False),
    )
    def k(table_hbm, idx_hbm, out_hbm, row_v, idx_v, out_v):
        wid = lax.axis_index("s") * nc + lax.axis_index("c")
        base = wid * per_w

        def do_row(j, _):
            row = base + j
            f = row // D
            pltpu.sync_copy(idx_hbm.at[f], idx_v)
            pltpu.sync_copy(table_hbm.at[row], row_v)

            def gather16(t, _):
                off = pl.multiple_of(t * 16, 16)
                idx16 = idx_v[pl.ds(off, 16)]
                out_v[pl.ds(off, 16)] = plsc.load_gather(row_v, [idx16])
                return 0

            lax.fori_loop(0, ngrp, gather16, 0)
            pltpu.sync_copy(out_v, out_hbm.at[row])
            return 0

        lax.fori_loop(0, per_w, do_row, 0)

    return k(table_t, idx2)


def _tc_body(xt_ref, lyt_ref, bw0, bb0, bw1, bb1, bw2, bb2,
             w0x, w0z, tb0, tw1, tb1, tw2, tb2, out_ref):
    f32 = jnp.float32
    x = xt_ref[...]
    h = jnp.maximum(jnp.dot(bw0[...], x, preferred_element_type=f32) + bb0[...], 0.0)
    h = jnp.maximum(jnp.dot(bw1[...], h, preferred_element_type=f32) + bb1[...], 0.0)
    x3 = jnp.maximum(jnp.dot(bw2[...], h, preferred_element_type=f32) + bb2[...], 0.0)

    t2 = jnp.concatenate([x3, lyt_ref[...]], axis=0)
    bb = t2.shape[1]
    t3 = t2.reshape(NF1, D, bb)
    zrows = []
    for i in range(NF1):
        zrows.append(jnp.sum(t3 * t3[i][None], axis=1))
    zt = jnp.concatenate(zrows, axis=0)

    a = jnp.dot(w0x[...], x3, preferred_element_type=f32)
    a = a + jnp.dot(w0z[...], zt, preferred_element_type=f32) + tb0[...]
    a = jnp.maximum(a, 0.0)
    a = jnp.maximum(jnp.dot(tw1[...], a, preferred_element_type=f32) + tb1[...], 0.0)
    o = jnp.dot(tw2[...], a, preferred_element_type=f32) + tb2[...]
    out_ref[...] = 1.0 / (1.0 + jnp.exp(-o))


def _tc_fused(xt, lyt, bw0, bb0, bw1, bb1, bw2, bb2,
              w0x, w0z, tb0, tw1, tb1, tw2, tb2, block_b=512):
    nb = B // block_b

    def full(a):
        return pl.BlockSpec(a.shape, lambda b: (0,) * a.ndim)

    weights = (bw0, bb0, bw1, bb1, bw2, bb2, w0x, w0z, tb0, tw1, tb1, tw2, tb2)
    return pl.pallas_call(
        _tc_body,
        grid=(nb,),
        in_specs=[
            pl.BlockSpec((13, block_b), lambda b: (0, b)),
            pl.BlockSpec((NFIELDS * D, block_b), lambda b: (0, b)),
        ] + [full(w) for w in weights],
        out_specs=pl.BlockSpec((1, block_b), lambda b: (0, b)),
        out_shape=jax.ShapeDtypeStruct((1, B), jnp.float32),
    )(xt, lyt, *weights)


def kernel(dense_x, lS_i, emb_tables, bot_w0, bot_b0, bot_w1, bot_b1,
           bot_w2, bot_b2, top_w0, top_b0, top_w1, top_b1, top_w2, top_b2):
    table_t = jnp.transpose(emb_tables, (0, 2, 1)).reshape(NFIELDS * D, VOCAB)

    lyt = _sc_gather(table_t, lS_i.astype(jnp.int32))
    xt = dense_x.T

    li, lj = np.tril_indices(NF1, -1)
    sel = np.asarray(li * NF1 + lj)
    w0x = top_w0[:, :D]
    w0z = jnp.zeros((top_w0.shape[0], ZDIM), jnp.float32).at[:, sel].set(top_w0[:, D:])

    def col(b):
        return b.reshape(-1, 1)

    out = _tc_fused(xt, lyt, bot_w0, col(bot_b0), bot_w1, col(bot_b1),
                    bot_w2, col(bot_b2), w0x, w0z, col(top_b0),
                    top_w1, col(top_b1), top_w2, col(top_b2))
    return out.reshape(B, 1)
